# baseline (speedup 1.0000x reference)
import functools

import jax
import jax.numpy as jnp
from jax import lax
from jax.experimental import pallas as pl
from jax.experimental.pallas import tpu as pltpu
from jax.experimental.pallas import tpu_sc as plsc

MAX_W = 10
T_PAD = 8224


def _attn_mlp_body(ce_ref, w1_ref, b1_ref, w2_ref, b2_ref, out_ref):
    h = jax.nn.relu(
        jnp.dot(ce_ref[...], w1_ref[...], preferred_element_type=jnp.float32)
        + b1_ref[...]
    )
    out_ref[...] = (
        jnp.dot(h, w2_ref[...], preferred_element_type=jnp.float32) + b2_ref[...]
    )


def _wsum_ffnn_body(win_ref, w_ref, se_ref, ee_ref, we_ref,
                    w1_ref, b1_ref, w2_ref, b2_ref, repr_ref, out_ref, *, a_dim):
    w = w_ref[...]
    ae = jnp.sum(win_ref[...] * w[:, :, None], axis=1)
    x = jnp.concatenate(
        [se_ref[:, :a_dim], ee_ref[:, :a_dim], ae, we_ref[...]], axis=1)
    repr_ref[...] = x
    h = jax.nn.relu(
        jnp.dot(x, w1_ref[...], preferred_element_type=jnp.float32) + b1_ref[...]
    )
    out_ref[...] = (
        jnp.dot(h, w2_ref[...], preferred_element_type=jnp.float32) + b2_ref[...]
    )


def _topk_body(s_ref, idx_ref):
    rows, cols = s_ref.shape
    nbits = (rows * cols).bit_length() - 1
    cbits = cols.bit_length() - 1
    s = s_ref[...]
    row_io = lax.broadcasted_iota(jnp.int32, (rows, cols), 0)
    lane_io = lax.broadcasted_iota(jnp.int32, (rows, cols), 1)
    pos = row_io * cols + lane_io
    x = pos

    def partner(v, jbit):
        if jbit < cbits:
            sh = 1 << jbit
            lo = pltpu.roll(v, cols - sh, 1)
            hi = pltpu.roll(v, sh, 1)
            m0 = (lane_io & sh) == 0
        else:
            sh = 1 << (jbit - cbits)
            lo = pltpu.roll(v, rows - sh, 0)
            hi = pltpu.roll(v, sh, 0)
            m0 = (row_io & sh) == 0
        return jnp.where(m0, lo, hi), m0

    for m in range(1, nbits + 1):
        for jbit in reversed(range(m)):
            ps, _ = partner(s, jbit)
            px, _ = partner(x, jbit)
            first = (s > ps) | ((s == ps) & (x < px))
            cond = (((pos >> jbit) ^ (pos >> m)) & 1) == 0
            s = jnp.where(cond, jnp.where(first, s, ps),
                          jnp.where(first, ps, s))
            x = jnp.where(cond, jnp.where(first, x, px),
                          jnp.where(first, px, x))
    idx_ref[...] = x


def _make_sc_gather(n, t_ctx, e_dim, a_dim, d_dim):
    nblk = n // 16
    mesh = plsc.VectorSubcoreMesh(core_axis_name="c", subcore_axis_name="s")

    @functools.partial(
        pl.kernel,
        mesh=mesh,
        out_type=[
            jax.ShapeDtypeStruct((n, 16), jnp.float32),
            jax.ShapeDtypeStruct((n, 128), jnp.float32),
            jax.ShapeDtypeStruct((n, 128), jnp.float32),
            jax.ShapeDtypeStruct((n, 16), jnp.float32),
            jax.ShapeDtypeStruct((n * MAX_W, e_dim), jnp.float32),
        ],
        compiler_params=pltpu.CompilerParams(needs_layout_passes=False),
        scratch_types=[
            pltpu.VMEM((16,), jnp.int32),
            pltpu.VMEM((16,), jnp.int32),
            pltpu.VMEM((16,), jnp.int32),
            pltpu.VMEM((16 * MAX_W,), jnp.int32),
            pltpu.VMEM((T_PAD,), jnp.float32),
            pltpu.VMEM((MAX_W * 16,), jnp.float32),
            pltpu.VMEM((16, 16), jnp.float32),
            pltpu.VMEM((16, 128), jnp.float32),
            pltpu.VMEM((16, 128), jnp.float32),
            pltpu.VMEM((16, 16), jnp.float32),
            pltpu.VMEM((16 * MAX_W, e_dim), jnp.float32),
            pltpu.SemaphoreType.DMA,
        ],
    )
    def sc_gather(attn_hbm, ctx_hbm, wt_hbm, emb_hbm, starts_hbm, widths_hbm,
                  sa_out, se_out, ee_out, we_out, win_out,
                  s_v, w_v, e_v, win_idx, attn_v, wt_v, sa_v,
                  se_rows, ee_rows, we_rows, win_rows, sem):
        nw = 32
        wid = lax.axis_index("s") * 2 + lax.axis_index("c")
        pltpu.sync_copy(attn_hbm, attn_v)
        pltpu.sync_copy(wt_hbm, wt_v)
        lanes = lax.iota(jnp.int32, 16)

        def body(t, carry):
            b = wid + nw * t

            @pl.when(b < nblk)
            def _():
                base = 16 * b
                pltpu.sync_copy(starts_hbm.at[pl.ds(base, 16)], s_v)
                pltpu.sync_copy(widths_hbm.at[pl.ds(base, 16)], w_v)
                s = s_v[...]
                w = w_v[...]
                e_v[...] = s + w
                for j in range(MAX_W):
                    plsc.store_scatter(win_idx, [lanes * MAX_W + j], s + j)
                cp_se = pltpu.async_copy(ctx_hbm.at[s_v], se_rows, sem)
                cp_ee = pltpu.async_copy(ctx_hbm.at[e_v], ee_rows, sem)
                cp_win = pltpu.async_copy(emb_hbm.at[win_idx], win_rows, sem)
                for j in range(16):
                    col = jnp.full((16,), j, jnp.int32)
                    a_j = plsc.load_gather(attn_v, [s + j])
                    plsc.store_scatter(sa_v, [lanes, col], a_j)
                    t_j = plsc.load_gather(wt_v, [w * 16 + j])
                    plsc.store_scatter(we_rows, [lanes, col], t_j)
                cp_se.wait()
                cp_ee.wait()
                cp_win.wait()
                pltpu.sync_copy(se_rows, se_out.at[pl.ds(base, 16)])
                pltpu.sync_copy(ee_rows, ee_out.at[pl.ds(base, 16)])
                pltpu.sync_copy(we_rows, we_out.at[pl.ds(base, 16)])
                pltpu.sync_copy(win_rows,
                                win_out.at[pl.ds(MAX_W * base, 16 * MAX_W)])
                pltpu.sync_copy(sa_v, sa_out.at[pl.ds(base, 16)])

            return carry

        lax.fori_loop(0, (nblk + nw - 1) // nw, body, 0)

    return sc_gather


def kernel(context_enc, embeds, span_starts, span_widths, attn_W1, attn_b1,
           attn_W2, attn_b2, width_table, ffnn_W1, ffnn_b1, ffnn_W2, ffnn_b2):
    Tn = context_enc.shape[0]
    E = embeds.shape[1]
    A = context_enc.shape[1]
    D = width_table.shape[1]
    n = span_starts.shape[0]

    attn_scores = pl.pallas_call(
        _attn_mlp_body,
        out_shape=jax.ShapeDtypeStruct((Tn, 1), jnp.float32),
    )(context_enc, attn_W1, attn_b1.reshape(1, -1), attn_W2, attn_b2.reshape(1, 1))

    attn_pad = jnp.concatenate(
        [attn_scores[:, 0], jnp.zeros((T_PAD - Tn,), jnp.float32)])

    ctx_pad = jnp.pad(context_enc, ((0, 0), (0, 128 - A)))
    sc_gather = _make_sc_gather(n, Tn, E, A, D)
    sa, start_emb, end_emb, width_emb, win = sc_gather(
        attn_pad, ctx_pad, width_table.reshape(-1), embeds,
        span_starts.astype(jnp.int32), span_widths.astype(jnp.int32))

    offsets = jnp.arange(MAX_W)
    mask = offsets[None, :] < (span_widths + 1)[:, None]
    span_attn = jnp.where(mask, sa[:, :MAX_W], -1e10)
    attn_weights = jax.nn.softmax(span_attn, axis=1)

    blk = 2000
    span_representations, mention_scores = pl.pallas_call(
        functools.partial(_wsum_ffnn_body, a_dim=A),
        grid=(n // blk,),
        in_specs=[
            pl.BlockSpec((blk, MAX_W, E), lambda i: (i, 0, 0)),
            pl.BlockSpec((blk, MAX_W), lambda i: (i, 0)),
            pl.BlockSpec((blk, 128), lambda i: (i, 0)),
            pl.BlockSpec((blk, 128), lambda i: (i, 0)),
            pl.BlockSpec((blk, D), lambda i: (i, 0)),
            pl.BlockSpec(ffnn_W1.shape, lambda i: (0, 0)),
            pl.BlockSpec((1, ffnn_b1.shape[0]), lambda i: (0, 0)),
            pl.BlockSpec(ffnn_W2.shape, lambda i: (0, 0)),
            pl.BlockSpec((1, 1), lambda i: (0, 0)),
        ],
        out_specs=[
            pl.BlockSpec((blk, 2 * A + E + D), lambda i: (i, 0)),
            pl.BlockSpec((blk, 1), lambda i: (i, 0)),
        ],
        out_shape=[
            jax.ShapeDtypeStruct((n, 2 * A + E + D), jnp.float32),
            jax.ShapeDtypeStruct((n, 1), jnp.float32),
        ],
    )(win.reshape(n, MAX_W, E), attn_weights, start_emb, end_emb,
      width_emb[:, :D], ffnn_W1, ffnn_b1.reshape(1, -1), ffnn_W2,
      ffnn_b2.reshape(1, 1))

    STOP = int(0.4 * Tn)
    k = min(STOP, mention_scores.shape[0])
    n_sort = 1 << (n - 1).bit_length()
    sc_pad = jnp.pad(mention_scores[:, 0], (0, n_sort - n),
                     constant_values=-jnp.inf).reshape(n_sort // 128, 128)
    sort_idx = pl.pallas_call(
        _topk_body,
        out_shape=jax.ShapeDtypeStruct(sc_pad.shape, jnp.int32),
    )(sc_pad)
    top_idx = sort_idx.reshape(-1)[:k]
    return (top_idx, span_representations, mention_scores)


# speedup vs baseline: 1.0629x; 1.0629x over previous
import functools

import jax
import jax.numpy as jnp
from jax import lax
from jax.experimental import pallas as pl
from jax.experimental.pallas import tpu as pltpu
from jax.experimental.pallas import tpu_sc as plsc

MAX_W = 10
T_PAD = 8224


def _attn_mlp_body(ce_ref, w1_ref, b1_ref, w2_ref, b2_ref, out_ref):
    h = jax.nn.relu(
        jnp.dot(ce_ref[...], w1_ref[...], preferred_element_type=jnp.float32)
        + b1_ref[...]
    )
    out_ref[...] = (
        jnp.dot(h, w2_ref[...], preferred_element_type=jnp.float32) + b2_ref[...]
    )


def _wsum_ffnn_body(win_ref, w_ref, se_ref, ee_ref, we_ref,
                    w1_ref, b1_ref, w2_ref, b2_ref, repr_ref, out_ref, *, a_dim):
    w = w_ref[...]
    ae = jnp.sum(win_ref[...] * w[:, :, None], axis=1)
    x = jnp.concatenate(
        [se_ref[:, :a_dim], ee_ref[:, :a_dim], ae, we_ref[...]], axis=1)
    repr_ref[...] = x
    h = jax.nn.relu(
        jnp.dot(x, w1_ref[...], preferred_element_type=jnp.float32) + b1_ref[...]
    )
    out_ref[...] = (
        jnp.dot(h, w2_ref[...], preferred_element_type=jnp.float32) + b2_ref[...]
    )


def _topk_body(s_ref, idx_ref):
    rows, cols = s_ref.shape
    nbits = (rows * cols).bit_length() - 1
    cbits = cols.bit_length() - 1
    s = s_ref[...]
    row_io = lax.broadcasted_iota(jnp.int32, (rows, cols), 0)
    lane_io = lax.broadcasted_iota(jnp.int32, (rows, cols), 1)
    pos = row_io * cols + lane_io
    x = pos

    def partner(v, jbit):
        if jbit < cbits:
            sh = 1 << jbit
            lo = pltpu.roll(v, cols - sh, 1)
            hi = pltpu.roll(v, sh, 1)
            m0 = (lane_io & sh) == 0
        else:
            sh = 1 << (jbit - cbits)
            lo = pltpu.roll(v, rows - sh, 0)
            hi = pltpu.roll(v, sh, 0)
            m0 = (row_io & sh) == 0
        return jnp.where(m0, lo, hi), m0

    for m in range(1, nbits + 1):
        for jbit in reversed(range(m)):
            ps, _ = partner(s, jbit)
            px, _ = partner(x, jbit)
            first = (s > ps) | ((s == ps) & (x < px))
            cond = (((pos >> jbit) ^ (pos >> m)) & 1) == 0
            s = jnp.where(cond, jnp.where(first, s, ps),
                          jnp.where(first, ps, s))
            x = jnp.where(cond, jnp.where(first, x, px),
                          jnp.where(first, px, x))
    idx_ref[...] = x


def _make_sc_gather(n_pad, e_dim):
    per_w = n_pad // 32
    nblk = per_w // 16
    mesh = plsc.VectorSubcoreMesh(core_axis_name="c", subcore_axis_name="s")

    @functools.partial(
        pl.kernel,
        mesh=mesh,
        out_type=[
            jax.ShapeDtypeStruct((n_pad, 16), jnp.float32),
            jax.ShapeDtypeStruct((n_pad, 128), jnp.float32),
            jax.ShapeDtypeStruct((n_pad, 128), jnp.float32),
            jax.ShapeDtypeStruct((n_pad, 16), jnp.float32),
            jax.ShapeDtypeStruct((n_pad * MAX_W, e_dim), jnp.float32),
        ],
        compiler_params=pltpu.CompilerParams(needs_layout_passes=False),
        scratch_types=[
            pltpu.VMEM((per_w,), jnp.int32),
            pltpu.VMEM((per_w,), jnp.int32),
            pltpu.VMEM((T_PAD,), jnp.float32),
            pltpu.VMEM((MAX_W * 16,), jnp.float32),
            pltpu.VMEM((32,), jnp.int32),
            pltpu.VMEM((32,), jnp.int32),
            pltpu.VMEM((16 * MAX_W,), jnp.int32),
            pltpu.VMEM((16 * MAX_W,), jnp.int32),
            pltpu.VMEM((32, 128), jnp.float32),
            pltpu.VMEM((32, 128), jnp.float32),
            pltpu.VMEM((16 * MAX_W, e_dim), jnp.float32),
            pltpu.VMEM((16 * MAX_W, e_dim), jnp.float32),
            pltpu.VMEM((16, 16), jnp.float32),
            pltpu.VMEM((16, 16), jnp.float32),
            pltpu.VMEM((16, 16), jnp.float32),
            pltpu.VMEM((16, 16), jnp.float32),
            pltpu.SemaphoreType.DMA,
            pltpu.SemaphoreType.DMA,
        ],
    )
    def sc_gather(attn_hbm, ctx_hbm, wt_hbm, emb_hbm, starts_hbm, widths_hbm,
                  sa_out, se_out, ee_out, we_out, win_out,
                  sv_all, wv_all, attn_v, wt_v, seee_idx0, seee_idx1,
                  win_idx0, win_idx1, seee_rows0, seee_rows1,
                  win_rows0, win_rows1, sa_v0, sa_v1, we_v0, we_v1,
                  sem_g, sem_o):
        seee_idx = (seee_idx0, seee_idx1)
        win_idx = (win_idx0, win_idx1)
        seee_rows = (seee_rows0, seee_rows1)
        win_rows = (win_rows0, win_rows1)
        sa_v = (sa_v0, sa_v1)
        we_v = (we_v0, we_v1)
        wid = lax.axis_index("s") * 2 + lax.axis_index("c")
        w0 = per_w * wid
        pltpu.sync_copy(starts_hbm.at[pl.ds(w0, per_w)], sv_all)
        pltpu.sync_copy(widths_hbm.at[pl.ds(w0, per_w)], wv_all)
        pltpu.sync_copy(attn_hbm, attn_v)
        pltpu.sync_copy(wt_hbm, wt_v)
        lanes = lax.iota(jnp.int32, 16)

        def prep_idx(tb):
            pb = tb % 2
            s = sv_all[pl.ds(16 * tb, 16)]
            w = wv_all[pl.ds(16 * tb, 16)]
            seee_idx[pb][pl.ds(0, 16)] = s
            seee_idx[pb][pl.ds(16, 16)] = s + w
            for j in range(MAX_W):
                plsc.store_scatter(win_idx[pb], [lanes * MAX_W + j], s + j)

        def issue_gathers(tb):
            pb = tb % 2
            g1 = pltpu.async_copy(ctx_hbm.at[seee_idx[pb]],
                                  seee_rows[pb], sem_g)
            g2 = pltpu.async_copy(emb_hbm.at[win_idx[pb]],
                                  win_rows[pb], sem_g)
            return (g1, g2)

        prep_idx(0)
        gh = issue_gathers(0)
        outs_prev = None
        for tb in range(nblk):
            pb = tb % 2
            for h in gh:
                h.wait()
            if tb + 1 < nblk:
                prep_idx(tb + 1)
                if outs_prev is not None:
                    for h in outs_prev:
                        h.wait()
                gh = issue_gathers(tb + 1)
            s = sv_all[pl.ds(16 * tb, 16)]
            w = wv_all[pl.ds(16 * tb, 16)]
            for j in range(MAX_W):
                col = jnp.full((16,), j, jnp.int32)
                a_j = plsc.load_gather(attn_v, [s + j])
                plsc.store_scatter(sa_v[pb], [lanes, col], a_j)
            for j in range(16):
                col = jnp.full((16,), j, jnp.int32)
                t_j = plsc.load_gather(wt_v, [w * 16 + j])
                plsc.store_scatter(we_v[pb], [lanes, col], t_j)
            base = w0 + 16 * tb
            outs = (
                pltpu.async_copy(seee_rows[pb].at[pl.ds(0, 16)],
                                 se_out.at[pl.ds(base, 16)], sem_o),
                pltpu.async_copy(seee_rows[pb].at[pl.ds(16, 16)],
                                 ee_out.at[pl.ds(base, 16)], sem_o),
                pltpu.async_copy(we_v[pb], we_out.at[pl.ds(base, 16)],
                                 sem_o),
                pltpu.async_copy(sa_v[pb], sa_out.at[pl.ds(base, 16)],
                                 sem_o),
                pltpu.async_copy(win_rows[pb],
                                 win_out.at[pl.ds(MAX_W * base, 16 * MAX_W)],
                                 sem_o),
            )
            outs_prev = outs
        for h in outs_prev:
            h.wait()

    return sc_gather


def kernel(context_enc, embeds, span_starts, span_widths, attn_W1, attn_b1,
           attn_W2, attn_b2, width_table, ffnn_W1, ffnn_b1, ffnn_W2, ffnn_b2):
    Tn = context_enc.shape[0]
    E = embeds.shape[1]
    A = context_enc.shape[1]
    D = width_table.shape[1]
    n = span_starts.shape[0]

    attn_scores = pl.pallas_call(
        _attn_mlp_body,
        out_shape=jax.ShapeDtypeStruct((Tn, 1), jnp.float32),
    )(context_enc, attn_W1, attn_b1.reshape(1, -1), attn_W2, attn_b2.reshape(1, 1))

    attn_pad = jnp.concatenate(
        [attn_scores[:, 0], jnp.zeros((T_PAD - Tn,), jnp.float32)])

    ctx_pad = jnp.pad(context_enc, ((0, 0), (0, 128 - A)))
    n_pad = 32 * 640
    starts_pad = jnp.pad(span_starts.astype(jnp.int32), (0, n_pad - n))
    widths_pad = jnp.pad(span_widths.astype(jnp.int32), (0, n_pad - n))
    sc_gather = _make_sc_gather(n_pad, E)
    sa, start_emb, end_emb, width_emb, win = sc_gather(
        attn_pad, ctx_pad, width_table.reshape(-1), embeds,
        starts_pad, widths_pad)

    offsets = jnp.arange(MAX_W)
    mask = offsets[None, :] < (span_widths + 1)[:, None]
    span_attn = jnp.where(mask, sa[:n, :MAX_W], -1e10)
    attn_weights = jax.nn.softmax(span_attn, axis=1)

    blk = 2000
    span_representations, mention_scores = pl.pallas_call(
        functools.partial(_wsum_ffnn_body, a_dim=A),
        grid=(n // blk,),
        in_specs=[
            pl.BlockSpec((blk, MAX_W, E), lambda i: (i, 0, 0)),
            pl.BlockSpec((blk, MAX_W), lambda i: (i, 0)),
            pl.BlockSpec((blk, 128), lambda i: (i, 0)),
            pl.BlockSpec((blk, 128), lambda i: (i, 0)),
            pl.BlockSpec((blk, D), lambda i: (i, 0)),
            pl.BlockSpec(ffnn_W1.shape, lambda i: (0, 0)),
            pl.BlockSpec((1, ffnn_b1.shape[0]), lambda i: (0, 0)),
            pl.BlockSpec(ffnn_W2.shape, lambda i: (0, 0)),
            pl.BlockSpec((1, 1), lambda i: (0, 0)),
        ],
        out_specs=[
            pl.BlockSpec((blk, 2 * A + E + D), lambda i: (i, 0)),
            pl.BlockSpec((blk, 1), lambda i: (i, 0)),
        ],
        out_shape=[
            jax.ShapeDtypeStruct((n, 2 * A + E + D), jnp.float32),
            jax.ShapeDtypeStruct((n, 1), jnp.float32),
        ],
    )(win.reshape(n_pad, MAX_W, E), attn_weights, start_emb, end_emb,
      width_emb, ffnn_W1, ffnn_b1.reshape(1, -1), ffnn_W2,
      ffnn_b2.reshape(1, 1))

    STOP = int(0.4 * Tn)
    k = min(STOP, mention_scores.shape[0])
    n_sort = 1 << (n - 1).bit_length()
    sc_pad = jnp.pad(mention_scores[:, 0], (0, n_sort - n),
                     constant_values=-jnp.inf).reshape(n_sort // 128, 128)
    sort_idx = pl.pallas_call(
        _topk_body,
        out_shape=jax.ShapeDtypeStruct(sc_pad.shape, jnp.int32),
    )(sc_pad)
    top_idx = sort_idx.reshape(-1)[:k]
    return (top_idx, span_representations, mention_scores)
